# baseline (speedup 1.0000x reference)
import functools

import jax
import jax.numpy as jnp
from jax import lax
from jax.experimental import pallas as pl
from jax.experimental.pallas import tpu as pltpu
from jax.experimental.pallas import tpu_sc as plsc

_VOCAB = 30522
_VOCAB_PAD = 30528
_EMB = 32
_B = 16384
_S = 200
_NW = 32
_BPW = _B // _NW
_G = _BPW // 16
_U = 8


def _table_body(embed_ref, w1_ref, b1_ref, wc_ref, bc_ref, out_ref):
    wf = jnp.dot(w1_ref[...], wc_ref[...], preferred_element_type=jnp.float32)
    bf = jnp.dot(b1_ref[...], wc_ref[...], preferred_element_type=jnp.float32)
    bf = bf + bc_ref[...]
    t = jnp.dot(embed_ref[...], wf, preferred_element_type=jnp.float32)
    out_ref[...] = (t + bf) * (1.0 / _S)


_table_kernel = pl.pallas_call(
    _table_body,
    out_shape=jax.ShapeDtypeStruct((_VOCAB_PAD, 2), jnp.float32),
)


def _sc_bag_body(tbl_hbm, texts_hbm, out_hbm, tbl_v, idx0, idx1,
                 out0_v, out1_v, sem_t, sem0, sem1):
    cid = lax.axis_index("c")
    sid = lax.axis_index("s")
    wid = sid * 2 + cid
    base = wid * _BPW

    tbl_copy = pltpu.async_copy(tbl_hbm, tbl_v, sem_t)
    bufs = (idx0, idx1)
    sems = (sem0, sem1)
    copies = [None, None]
    copies[0] = pltpu.async_copy(
        texts_hbm.at[pl.ds(base, 16), :], idx0, sem0)
    tbl_copy.wait()

    lanes = lax.iota(jnp.int32, 16)

    for g in range(_G):
        cur = g & 1
        if g + 1 < _G:
            nxt = (g + 1) & 1
            copies[nxt] = pltpu.async_copy(
                texts_hbm.at[pl.ds(base + (g + 1) * 16, 16), :],
                bufs[nxt], sems[nxt])
        copies[cur].wait()
        iref = bufs[cur]

        def body(i, carry, iref=iref):
            a0, a1 = carry
            for j in range(_U):
                t = i * _U + j
                tv = jnp.broadcast_to(t, (16,)).astype(jnp.int32)
                iv = plsc.load_gather(iref, [lanes, tv])
                o = iv * 2
                v0 = plsc.load_gather(tbl_v, [o])
                v1 = plsc.load_gather(tbl_v, [o + 1])
                a0 = a0 + v0
                a1 = a1 + v1
            return (a0, a1)

        zero = jnp.zeros((16,), jnp.float32)
        acc0, acc1 = lax.fori_loop(0, _S // _U, body, (zero, zero))
        out0_v[pl.ds(g * 16, 16)] = acc0
        out1_v[pl.ds(g * 16, 16)] = acc1

    pltpu.sync_copy(out0_v, out_hbm.at[pl.ds(base, _BPW)])
    pltpu.sync_copy(out1_v, out_hbm.at[pl.ds(_B + base, _BPW)])


_sc_bag = functools.partial(
    pl.kernel,
    out_type=jax.ShapeDtypeStruct((2 * _B,), jnp.float32),
    mesh=plsc.VectorSubcoreMesh(core_axis_name="c", subcore_axis_name="s"),
    compiler_params=pltpu.CompilerParams(needs_layout_passes=False),
    scratch_types=[
        pltpu.VMEM((2 * _VOCAB_PAD,), jnp.float32),
        pltpu.VMEM((16, _S), jnp.int32),
        pltpu.VMEM((16, _S), jnp.int32),
        pltpu.VMEM((_BPW,), jnp.float32),
        pltpu.VMEM((_BPW,), jnp.float32),
        pltpu.SemaphoreType.DMA,
        pltpu.SemaphoreType.DMA,
        pltpu.SemaphoreType.DMA,
    ],
)(_sc_bag_body)


def kernel(texts, embed, W1, b1, Wc, bc):
    embed_pad = jnp.pad(embed, ((0, _VOCAB_PAD - _VOCAB), (0, 0)))
    tbl = _table_kernel(embed_pad, W1, b1.reshape(1, -1), Wc,
                        bc.reshape(1, -1))
    tbl_flat = tbl.reshape(-1)
    out = _sc_bag(tbl_flat, texts)
    return out.reshape(2, _B).T


# speedup vs baseline: 1.1490x; 1.1490x over previous
import functools

import jax
import jax.numpy as jnp
from jax import lax
from jax.experimental import pallas as pl
from jax.experimental.pallas import tpu as pltpu
from jax.experimental.pallas import tpu_sc as plsc

_VOCAB = 30522
_VOCAB_PAD = 30528
_EMB = 32
_B = 16384
_S = 200
_NW = 32
_BPW = _B // _NW
_G = _BPW // 16
_U = 8


def _table_body(embed_ref, w1_ref, b1_ref, wc_ref, bc_ref, out_ref):
    wf = jnp.dot(w1_ref[...], wc_ref[...], preferred_element_type=jnp.float32)
    bf = jnp.dot(b1_ref[...], wc_ref[...], preferred_element_type=jnp.float32)
    bf = bf + bc_ref[...]
    t = jnp.dot(embed_ref[...], wf, preferred_element_type=jnp.float32)
    out_ref[...] = (t + bf) * (1.0 / _S)


_table_kernel = pl.pallas_call(
    _table_body,
    out_shape=jax.ShapeDtypeStruct((_VOCAB_PAD, 2), jnp.float32),
)


def _sc_bag_body(tbl_hbm, texts_hbm, out_hbm, tbl_v, idx0, idx1,
                 out0_v, out1_v, sem_t, sem0, sem1):
    cid = lax.axis_index("c")
    sid = lax.axis_index("s")
    wid = sid * 2 + cid
    base = wid * _BPW

    tbl_copy = pltpu.async_copy(tbl_hbm, tbl_v, sem_t)
    bufs = (idx0, idx1)
    sems = (sem0, sem1)
    copies = [None, None]
    gbase = wid * _G
    copies[0] = pltpu.async_copy(texts_hbm.at[gbase], idx0, sem0)
    tbl_copy.wait()

    offs = lax.iota(jnp.int32, 16) * _S

    for g in range(_G):
        cur = g & 1
        if g + 1 < _G:
            nxt = (g + 1) & 1
            copies[nxt] = pltpu.async_copy(
                texts_hbm.at[gbase + g + 1], bufs[nxt], sems[nxt])
        copies[cur].wait()
        iref = bufs[cur]

        def body(i, carry, iref=iref):
            a0, a1 = carry
            for j in range(_U):
                t = i * _U + j
                tv = offs + jnp.broadcast_to(t, (16,)).astype(jnp.int32)
                iv = plsc.load_gather(iref, [tv])
                o = iv * 2
                v0 = plsc.load_gather(tbl_v, [o])
                v1 = plsc.load_gather(tbl_v, [o + 1])
                a0 = a0 + v0
                a1 = a1 + v1
            return (a0, a1)

        zero = jnp.zeros((16,), jnp.float32)
        acc0, acc1 = lax.fori_loop(0, _S // _U, body, (zero, zero))
        out0_v[pl.ds(g * 16, 16)] = acc0
        out1_v[pl.ds(g * 16, 16)] = acc1

    pltpu.sync_copy(out0_v, out_hbm.at[pl.ds(base, _BPW)])
    pltpu.sync_copy(out1_v, out_hbm.at[pl.ds(_B + base, _BPW)])


_sc_bag = functools.partial(
    pl.kernel,
    out_type=jax.ShapeDtypeStruct((2 * _B,), jnp.float32),
    mesh=plsc.VectorSubcoreMesh(core_axis_name="c", subcore_axis_name="s"),
    compiler_params=pltpu.CompilerParams(needs_layout_passes=False),
    scratch_types=[
        pltpu.VMEM((2 * _VOCAB_PAD,), jnp.float32),
        pltpu.VMEM((16 * _S,), jnp.int32),
        pltpu.VMEM((16 * _S,), jnp.int32),
        pltpu.VMEM((_BPW,), jnp.float32),
        pltpu.VMEM((_BPW,), jnp.float32),
        pltpu.SemaphoreType.DMA,
        pltpu.SemaphoreType.DMA,
        pltpu.SemaphoreType.DMA,
    ],
)(_sc_bag_body)


def kernel(texts, embed, W1, b1, Wc, bc):
    embed_pad = jnp.pad(embed, ((0, _VOCAB_PAD - _VOCAB), (0, 0)))
    tbl = _table_kernel(embed_pad, W1, b1.reshape(1, -1), Wc,
                        bc.reshape(1, -1))
    tbl_flat = tbl.reshape(-1)
    out = _sc_bag(tbl_flat, texts.reshape(_B // 16, 16 * _S))
    return out.reshape(2, _B).T


# speedup vs baseline: 1.4030x; 1.2211x over previous
import functools

import jax
import jax.numpy as jnp
from jax import lax
from jax.experimental import pallas as pl
from jax.experimental.pallas import tpu as pltpu
from jax.experimental.pallas import tpu_sc as plsc

_VOCAB = 30522
_B = 16384
_S = 200
_NW = 32
_BPW = _B // _NW
_G = _BPW // 16
_U = 8
_TBLK = 4096
_TLEN = 32768

_HIGHEST = jax.lax.Precision.HIGHEST


def _table_body(embed_ref, w1_ref, b1col_ref, wc_ref, bccol_ref, out_ref):
    wf = jax.lax.dot_general(
        w1_ref[...], wc_ref[...], (((1,), (0,)), ((), ())),
        preferred_element_type=jnp.float32, precision=_HIGHEST)
    bf = jax.lax.dot_general(
        wc_ref[...], b1col_ref[...], (((0,), (0,)), ((), ())),
        preferred_element_type=jnp.float32, precision=_HIGHEST)
    bf = bf + bccol_ref[...]
    tt = jax.lax.dot_general(
        wf, embed_ref[...], (((0,), (1,)), ((), ())),
        preferred_element_type=jnp.float32, precision=_HIGHEST)
    tt = (tt + bf) * (1.0 / _S)
    u0 = jax.lax.bitcast_convert_type(
        tt[0:1, :].astype(jnp.bfloat16), jnp.uint16).astype(jnp.int32)
    u1 = jax.lax.bitcast_convert_type(
        tt[1:2, :].astype(jnp.bfloat16), jnp.uint16).astype(jnp.int32)
    out_ref[...] = jnp.reshape(u0 | (u1 << 16), (_TBLK,))


_table_kernel = pl.pallas_call(
    _table_body,
    grid=(_TLEN // _TBLK,),
    in_specs=[
        pl.BlockSpec((_TBLK, 32), lambda i: (i, 0)),
        pl.BlockSpec((32, 128), lambda i: (0, 0)),
        pl.BlockSpec((128, 1), lambda i: (0, 0)),
        pl.BlockSpec((128, 2), lambda i: (0, 0)),
        pl.BlockSpec((2, 1), lambda i: (0, 0)),
    ],
    out_specs=pl.BlockSpec((_TBLK,), lambda i: (i,)),
    out_shape=jax.ShapeDtypeStruct((_TLEN,), jnp.int32),
)


def _sc_bag_body(tbl_hbm, texts_hbm, out_hbm, tbl_v, idx0, idx1,
                 out0_v, out1_v, sem_t, sem0, sem1):
    cid = lax.axis_index("c")
    sid = lax.axis_index("s")
    wid = sid * 2 + cid
    base = wid * _BPW

    tbl_copy = pltpu.async_copy(tbl_hbm, tbl_v, sem_t)
    bufs = (idx0, idx1)
    sems = (sem0, sem1)
    copies = [None, None]
    gbase = wid * _G
    copies[0] = pltpu.async_copy(texts_hbm.at[gbase], idx0, sem0)
    tbl_copy.wait()

    offs = lax.iota(jnp.int32, 16) * _S

    for g in range(_G):
        cur = g & 1
        if g + 1 < _G:
            nxt = (g + 1) & 1
            copies[nxt] = pltpu.async_copy(
                texts_hbm.at[gbase + g + 1], bufs[nxt], sems[nxt])
        copies[cur].wait()
        iref = bufs[cur]

        def body(i, carry, iref=iref):
            a0, a1 = carry
            for j in range(_U):
                t = i * _U + j
                tv = offs + jnp.broadcast_to(t, (16,)).astype(jnp.int32)
                iv = plsc.load_gather(iref, [tv])
                w = plsc.load_gather(tbl_v, [iv])
                pair = plsc.bitcast(w, jnp.bfloat16)
                v0, v1 = plsc.unpack(pair, format=plsc.PackFormat.INTERLEAVED)
                a0 = a0 + v0
                a1 = a1 + v1
            return (a0, a1)

        zero = jnp.zeros((16,), jnp.float32)
        acc0, acc1 = lax.fori_loop(0, _S // _U, body, (zero, zero))
        out0_v[pl.ds(g * 16, 16)] = acc0
        out1_v[pl.ds(g * 16, 16)] = acc1

    pltpu.sync_copy(out0_v, out_hbm.at[pl.ds(base, _BPW)])
    pltpu.sync_copy(out1_v, out_hbm.at[pl.ds(_B + base, _BPW)])


_sc_bag = functools.partial(
    pl.kernel,
    out_type=jax.ShapeDtypeStruct((2 * _B,), jnp.float32),
    mesh=plsc.VectorSubcoreMesh(core_axis_name="c", subcore_axis_name="s"),
    compiler_params=pltpu.CompilerParams(needs_layout_passes=False),
    scratch_types=[
        pltpu.VMEM((_TLEN,), jnp.int32),
        pltpu.VMEM((16 * _S,), jnp.int32),
        pltpu.VMEM((16 * _S,), jnp.int32),
        pltpu.VMEM((_BPW,), jnp.float32),
        pltpu.VMEM((_BPW,), jnp.float32),
        pltpu.SemaphoreType.DMA,
        pltpu.SemaphoreType.DMA,
        pltpu.SemaphoreType.DMA,
    ],
)(_sc_bag_body)


def kernel(texts, embed, W1, b1, Wc, bc):
    tbl = _table_kernel(embed, W1, b1.reshape(-1, 1), Wc, bc.reshape(-1, 1))
    out = _sc_bag(tbl, texts.reshape(_B // 16, 16 * _S))
    return out.reshape(2, _B).T
